# Initial kernel scaffold; baseline (speedup 1.0000x reference)
import jax
import jax.numpy as jnp
from jax.experimental import pallas as pl


def kernel(x, edge_index, W1, b1, W2, b2, W3, b3, Wi, bi, Wc, bc):
    raise NotImplementedError("write your pallas kernel here")


# speedup vs baseline: 14.3654x; 14.3654x over previous
import functools

import jax
import jax.numpy as jnp
from jax import lax
from jax.experimental import pallas as pl
from jax.experimental.pallas import tpu as pltpu
from jax.experimental.pallas import tpu_sc as plsc

N = 10000
E = 320000
D = 128
D_INT = 512

NC = 2
NS = 16
NW = NC * NS
STRIPE = 640
NPAD = NS * STRIPE
EPT = E // NW
C = 80
ITERS = EPT // C

_mesh = plsc.VectorSubcoreMesh(
    core_axis_name="c", subcore_axis_name="s", num_cores=NC, num_subcores=NS)

_Z16 = functools.partial(jnp.zeros, (16,), jnp.float32)


def _deg_body(dst_hbm, out_hbm, didx, ones_v, zbuf, deg_sh):
    c = lax.axis_index("c")
    s = lax.axis_index("s")

    def fill_ones(k, carry):
        ones_v[pl.ds(k * 16, 16)] = jnp.full((16,), 1.0, jnp.float32)
        return carry
    lax.fori_loop(0, C // 16, fill_ones, 0)

    def fill_zero(k, carry):
        zbuf[pl.ds(k * 16, 16)] = _Z16()
        return carry
    lax.fori_loop(0, STRIPE // 16, fill_zero, 0)

    pltpu.sync_copy(zbuf, deg_sh.at[pl.ds(s * STRIPE, STRIPE)])
    plsc.subcore_barrier()

    base0 = (c * NS + s) * EPT

    def step(i, carry):
        pltpu.sync_copy(dst_hbm.at[pl.ds(base0 + i * C, C)], didx)
        pltpu.sync_copy(ones_v, deg_sh.at[didx], add=True)
        return carry
    lax.fori_loop(0, ITERS, step, 0)

    plsc.subcore_barrier()
    pltpu.sync_copy(deg_sh.at[pl.ds(s * STRIPE, STRIPE)],
                    out_hbm.at[pl.ds(c * NPAD + s * STRIPE, STRIPE)])


_deg_call = pl.kernel(
    _deg_body,
    out_type=jax.ShapeDtypeStruct((NC * NPAD,), jnp.float32),
    mesh=_mesh,
    scratch_types=[
        pltpu.VMEM((C,), jnp.int32),
        pltpu.VMEM((C,), jnp.float32),
        pltpu.VMEM((STRIPE,), jnp.float32),
        pltpu.VMEM_SHARED((NPAD,), jnp.float32),
    ],
)


def _agg_body(g_hbm, src_hbm, dst_hbm, out_hbm, sidx, didx, rows, sem, acc):
    c = lax.axis_index("c")
    s = lax.axis_index("s")

    def zrow(j, carry):
        def zcol(k, carry2):
            rows[j, pl.ds(k * 16, 16)] = _Z16()
            return carry2
        return lax.fori_loop(0, D // 16, zcol, carry)
    lax.fori_loop(0, C, zrow, 0)

    def zstripe(m, carry):
        pltpu.sync_copy(rows, acc.at[pl.ds(s * STRIPE + m * C, C)])
        return carry
    lax.fori_loop(0, STRIPE // C, zstripe, 0)
    plsc.subcore_barrier()

    base0 = (c * NS + s) * EPT

    def step(i, carry):
        b = base0 + i * C
        pltpu.sync_copy(src_hbm.at[pl.ds(b, C)], sidx)
        pltpu.sync_copy(dst_hbm.at[pl.ds(b, C)], didx)
        pltpu.async_copy(g_hbm.at[sidx], rows, sem).wait()
        pltpu.sync_copy(rows, acc.at[didx], add=True)
        return carry
    lax.fori_loop(0, ITERS, step, 0)

    plsc.subcore_barrier()
    pltpu.sync_copy(acc.at[pl.ds(s * STRIPE, STRIPE)],
                    out_hbm.at[pl.ds(c * NPAD + s * STRIPE, STRIPE)])


_agg_call = pl.kernel(
    _agg_body,
    out_type=jax.ShapeDtypeStruct((NC * NPAD, D), jnp.float32),
    mesh=_mesh,
    scratch_types=[
        pltpu.VMEM((C,), jnp.int32),
        pltpu.VMEM((C,), jnp.int32),
        pltpu.VMEM((C, D), jnp.float32),
        pltpu.SemaphoreType.DMA,
        pltpu.VMEM_SHARED((NPAD, D), jnp.float32),
    ],
)


R = 1000
G = N // R


def _scale_body(x_ref, dinv_ref, o_ref):
    o_ref[...] = x_ref[...] * dinv_ref[...]


_scale = pl.pallas_call(
    _scale_body,
    grid=(G,),
    in_specs=[
        pl.BlockSpec((R, D), lambda i: (i, 0)),
        pl.BlockSpec((R, 1), lambda i: (i, 0)),
    ],
    out_specs=pl.BlockSpec((R, D), lambda i: (i, 0)),
    out_shape=jax.ShapeDtypeStruct((N, D), jnp.float32),
)


def _layer_body(aggp_ref, g_ref, dinv_ref, w_ref, b_ref, o_ref):
    dv = dinv_ref[...]
    t = (aggp_ref[0] + aggp_ref[1] + g_ref[...]) * dv
    h = jnp.dot(t, w_ref[...], preferred_element_type=jnp.float32) + b_ref[...]
    o_ref[...] = jnp.maximum(h, 0.0) * dv


_layer = pl.pallas_call(
    _layer_body,
    grid=(G,),
    in_specs=[
        pl.BlockSpec((NC, R, D), lambda i: (0, i, 0)),
        pl.BlockSpec((R, D), lambda i: (i, 0)),
        pl.BlockSpec((R, 1), lambda i: (i, 0)),
        pl.BlockSpec((D, D), lambda i: (0, 0)),
        pl.BlockSpec((1, D), lambda i: (0, 0)),
    ],
    out_specs=pl.BlockSpec((R, D), lambda i: (i, 0)),
    out_shape=jax.ShapeDtypeStruct((N, D), jnp.float32),
)


def _head_body(aggp_ref, g_ref, dinv_ref, w3_ref, b3_ref, wi_ref, bi_ref,
               wc_ref, bc_ref, o_ref):
    t = (aggp_ref[0] + aggp_ref[1] + g_ref[...]) * dinv_ref[...]
    h3 = jnp.dot(t, w3_ref[...], preferred_element_type=jnp.float32) + b3_ref[...]
    h3 = jnp.maximum(h3, 0.0)
    h4 = jnp.dot(h3, wi_ref[...], preferred_element_type=jnp.float32) + bi_ref[...]
    h4 = jnp.maximum(h4, 0.0)
    o_ref[...] = jnp.dot(h4, wc_ref[...], preferred_element_type=jnp.float32) + bc_ref[...]


_head = pl.pallas_call(
    _head_body,
    grid=(G,),
    in_specs=[
        pl.BlockSpec((NC, R, D), lambda i: (0, i, 0)),
        pl.BlockSpec((R, D), lambda i: (i, 0)),
        pl.BlockSpec((R, 1), lambda i: (i, 0)),
        pl.BlockSpec((D, D_INT), lambda i: (0, 0)),
        pl.BlockSpec((1, D_INT), lambda i: (0, 0)),
        pl.BlockSpec((D_INT, D_INT), lambda i: (0, 0)),
        pl.BlockSpec((1, D_INT), lambda i: (0, 0)),
        pl.BlockSpec((D_INT, D), lambda i: (0, 0)),
        pl.BlockSpec((1, D), lambda i: (0, 0)),
    ],
    out_specs=pl.BlockSpec((R, D), lambda i: (i, 0)),
    out_shape=jax.ShapeDtypeStruct((N, D), jnp.float32),
)


def kernel(x, edge_index, W1, b1, W2, b2, W3, b3, Wi, bi, Wc, bc):
    src = edge_index[0]
    dst = edge_index[1]

    degp = _deg_call(dst).reshape(NC, NPAD)
    deg = degp[0, :N] + degp[1, :N] + 1.0
    dinv = lax.rsqrt(deg).reshape(N, 1)

    g = _scale(x, dinv)
    aggp = _agg_call(g, src, dst).reshape(NC, NPAD, D)
    g = _layer(aggp, g, dinv, W1, b1.reshape(1, D))
    aggp = _agg_call(g, src, dst).reshape(NC, NPAD, D)
    g = _layer(aggp, g, dinv, W2, b2.reshape(1, D))
    aggp = _agg_call(g, src, dst).reshape(NC, NPAD, D)

    wc_pad = jnp.pad(Wc, ((0, 0), (0, D - Wc.shape[1])))
    bc_pad = jnp.pad(bc, (0, D - bc.shape[0])).reshape(1, D)
    out = _head(aggp, g, dinv, W3, b3.reshape(1, D_INT),
                Wi, bi.reshape(1, D_INT), wc_pad, bc_pad)
    return out[:, :Wc.shape[1]]


# speedup vs baseline: 14.8464x; 1.0335x over previous
import functools

import jax
import jax.numpy as jnp
from jax import lax
from jax.experimental import pallas as pl
from jax.experimental.pallas import tpu as pltpu
from jax.experimental.pallas import tpu_sc as plsc

N = 10000
E = 320000
D = 128
D_INT = 512

NC = 2
NS = 16
NW = NC * NS
STRIPE = 640
NPAD = NS * STRIPE
EPT = E // NW
C = 80
ITERS = EPT // C

_mesh = plsc.VectorSubcoreMesh(
    core_axis_name="c", subcore_axis_name="s", num_cores=NC, num_subcores=NS)

_Z16 = functools.partial(jnp.zeros, (16,), jnp.float32)


def _deg_body(dst_hbm, out_hbm, didx_all, ones_v, zbuf, deg_sh):
    c = lax.axis_index("c")
    s = lax.axis_index("s")

    def fill_ones(k, carry):
        ones_v[pl.ds(k * 16, 16)] = jnp.full((16,), 1.0, jnp.float32)
        return carry
    lax.fori_loop(0, C // 16, fill_ones, 0)

    def fill_zero(k, carry):
        zbuf[pl.ds(k * 16, 16)] = _Z16()
        return carry
    lax.fori_loop(0, STRIPE // 16, fill_zero, 0)

    pltpu.sync_copy(dst_hbm.at[c * NS + s], didx_all)
    pltpu.sync_copy(zbuf, deg_sh.at[pl.ds(s * STRIPE, STRIPE)])
    plsc.subcore_barrier()

    def step(i, carry):
        pltpu.sync_copy(ones_v, deg_sh.at[didx_all.at[i]], add=True)
        return carry
    lax.fori_loop(0, ITERS, step, 0)

    plsc.subcore_barrier()
    pltpu.sync_copy(deg_sh.at[pl.ds(s * STRIPE, STRIPE)],
                    out_hbm.at[pl.ds(c * NPAD + s * STRIPE, STRIPE)])


_deg_call = pl.kernel(
    _deg_body,
    out_type=jax.ShapeDtypeStruct((NC * NPAD,), jnp.float32),
    mesh=_mesh,
    scratch_types=[
        pltpu.VMEM((ITERS, C), jnp.int32),
        pltpu.VMEM((C,), jnp.float32),
        pltpu.VMEM((STRIPE,), jnp.float32),
        pltpu.VMEM_SHARED((NPAD,), jnp.float32),
    ],
)


def _agg_body(g_hbm, idx_hbm, out_hbm, ib0, ib1, rows0, rows1, sem, acc):
    c = lax.axis_index("c")
    s = lax.axis_index("s")
    wid = c * NS + s

    def zrow(j, carry):
        def zcol(k, carry2):
            rows0[j, pl.ds(k * 16, 16)] = _Z16()
            return carry2
        return lax.fori_loop(0, D // 16, zcol, carry)
    lax.fori_loop(0, C, zrow, 0)

    def zstripe(m, carry):
        pltpu.sync_copy(rows0, acc.at[pl.ds(s * STRIPE + m * C, C)])
        return carry
    lax.fori_loop(0, STRIPE // C, zstripe, 0)
    plsc.subcore_barrier()

    pltpu.sync_copy(idx_hbm.at[wid, 0], ib0)
    pltpu.async_copy(g_hbm.at[ib0.at[0]], rows0, sem)

    def step(i, carry):
        k = 2 * i
        pltpu.sync_copy(idx_hbm.at[wid, k + 1], ib1)
        pltpu.make_async_copy(g_hbm.at[ib0.at[0]], rows0, sem).wait()
        pltpu.async_copy(g_hbm.at[ib1.at[0]], rows1, sem)
        pltpu.sync_copy(rows0, acc.at[ib0.at[1]], add=True)
        pltpu.sync_copy(idx_hbm.at[wid, k + 2], ib0)
        pltpu.make_async_copy(g_hbm.at[ib1.at[0]], rows1, sem).wait()
        pltpu.async_copy(g_hbm.at[ib0.at[0]], rows0, sem)
        pltpu.sync_copy(rows1, acc.at[ib1.at[1]], add=True)
        return carry
    lax.fori_loop(0, (ITERS - 1) // 2, step, 0)

    pltpu.make_async_copy(g_hbm.at[ib0.at[0]], rows0, sem).wait()
    pltpu.sync_copy(rows0, acc.at[ib0.at[1]], add=True)

    plsc.subcore_barrier()
    pltpu.sync_copy(acc.at[pl.ds(s * STRIPE, STRIPE)],
                    out_hbm.at[pl.ds(c * NPAD + s * STRIPE, STRIPE)])


_agg_call = pl.kernel(
    _agg_body,
    out_type=jax.ShapeDtypeStruct((NC * NPAD, D), jnp.float32),
    mesh=_mesh,
    scratch_types=[
        pltpu.VMEM((2, C), jnp.int32),
        pltpu.VMEM((2, C), jnp.int32),
        pltpu.VMEM((C, D), jnp.float32),
        pltpu.VMEM((C, D), jnp.float32),
        pltpu.SemaphoreType.DMA,
        pltpu.VMEM_SHARED((NPAD, D), jnp.float32),
    ],
)


R = 1000
G = N // R


def _pre_body(x_ref, dinv_ref, w_ref, o_ref):
    xw = jnp.dot(x_ref[...], w_ref[...], preferred_element_type=jnp.float32)
    o_ref[...] = xw * dinv_ref[...]


_pre = pl.pallas_call(
    _pre_body,
    grid=(G,),
    in_specs=[
        pl.BlockSpec((R, D), lambda i: (i, 0)),
        pl.BlockSpec((R, 1), lambda i: (i, 0)),
        pl.BlockSpec((D, D), lambda i: (0, 0)),
    ],
    out_specs=pl.BlockSpec((R, D), lambda i: (i, 0)),
    out_shape=jax.ShapeDtypeStruct((N, D), jnp.float32),
)


def _mid_body(aggp_ref, q_ref, dinv_ref, b_ref, w_ref, o_ref):
    dv = dinv_ref[...]
    h = (aggp_ref[0] + aggp_ref[1] + q_ref[...]) * dv + b_ref[...]
    h = jnp.maximum(h, 0.0)
    o_ref[...] = jnp.dot(h, w_ref[...], preferred_element_type=jnp.float32) * dv


_mid = pl.pallas_call(
    _mid_body,
    grid=(G,),
    in_specs=[
        pl.BlockSpec((NC, R, D), lambda i: (0, i, 0)),
        pl.BlockSpec((R, D), lambda i: (i, 0)),
        pl.BlockSpec((R, 1), lambda i: (i, 0)),
        pl.BlockSpec((1, D), lambda i: (0, 0)),
        pl.BlockSpec((D, D), lambda i: (0, 0)),
    ],
    out_specs=pl.BlockSpec((R, D), lambda i: (i, 0)),
    out_shape=jax.ShapeDtypeStruct((N, D), jnp.float32),
)


def _mid3_body(aggp_ref, q_ref, dinv_ref, b_ref, w_ref,
               o0_ref, o1_ref, o2_ref, o3_ref):
    dv = dinv_ref[...]
    h = (aggp_ref[0] + aggp_ref[1] + q_ref[...]) * dv + b_ref[...]
    h = jnp.maximum(h, 0.0)
    q3 = jnp.dot(h, w_ref[...], preferred_element_type=jnp.float32) * dv
    o0_ref[...] = q3[:, :D]
    o1_ref[...] = q3[:, D:2 * D]
    o2_ref[...] = q3[:, 2 * D:3 * D]
    o3_ref[...] = q3[:, 3 * D:]


_mid3 = pl.pallas_call(
    _mid3_body,
    grid=(G,),
    in_specs=[
        pl.BlockSpec((NC, R, D), lambda i: (0, i, 0)),
        pl.BlockSpec((R, D), lambda i: (i, 0)),
        pl.BlockSpec((R, 1), lambda i: (i, 0)),
        pl.BlockSpec((1, D), lambda i: (0, 0)),
        pl.BlockSpec((D, D_INT), lambda i: (0, 0)),
    ],
    out_specs=tuple(pl.BlockSpec((R, D), lambda i: (i, 0)) for _ in range(4)),
    out_shape=tuple(jax.ShapeDtypeStruct((N, D), jnp.float32) for _ in range(4)),
)


def _head_body(a0_ref, a1_ref, a2_ref, a3_ref, q0_ref, q1_ref, q2_ref, q3_ref,
               dinv_ref, b3_ref, wi_ref, bi_ref, wc_ref, bc_ref, o_ref):
    dv = dinv_ref[...]
    t = jnp.concatenate(
        [(a0_ref[0] + a0_ref[1] + q0_ref[...]),
         (a1_ref[0] + a1_ref[1] + q1_ref[...]),
         (a2_ref[0] + a2_ref[1] + q2_ref[...]),
         (a3_ref[0] + a3_ref[1] + q3_ref[...])], axis=1) * dv
    h3 = jnp.maximum(t + b3_ref[...], 0.0)
    h4 = jnp.dot(h3, wi_ref[...], preferred_element_type=jnp.float32) + bi_ref[...]
    h4 = jnp.maximum(h4, 0.0)
    o_ref[...] = jnp.dot(h4, wc_ref[...], preferred_element_type=jnp.float32) + bc_ref[...]


_head = pl.pallas_call(
    _head_body,
    grid=(G,),
    in_specs=(
        [pl.BlockSpec((NC, R, D), lambda i: (0, i, 0)) for _ in range(4)]
        + [pl.BlockSpec((R, D), lambda i: (i, 0)) for _ in range(4)]
        + [
            pl.BlockSpec((R, 1), lambda i: (i, 0)),
            pl.BlockSpec((1, D_INT), lambda i: (0, 0)),
            pl.BlockSpec((D_INT, D_INT), lambda i: (0, 0)),
            pl.BlockSpec((1, D_INT), lambda i: (0, 0)),
            pl.BlockSpec((D_INT, D), lambda i: (0, 0)),
            pl.BlockSpec((1, D), lambda i: (0, 0)),
        ]
    ),
    out_specs=pl.BlockSpec((R, D), lambda i: (i, 0)),
    out_shape=jax.ShapeDtypeStruct((N, D), jnp.float32),
)


def kernel(x, edge_index, W1, b1, W2, b2, W3, b3, Wi, bi, Wc, bc):
    dst = edge_index[1].reshape(NW, ITERS, C)
    idx = edge_index.reshape(2, NW, ITERS, C).transpose(1, 2, 0, 3)

    degp = _deg_call(dst).reshape(NC, NPAD)
    deg = degp[0, :N] + degp[1, :N] + 1.0
    dinv = lax.rsqrt(deg).reshape(N, 1)

    q = _pre(x, dinv, W1)
    aggp = _agg_call(q, idx).reshape(NC, NPAD, D)
    q = _mid(aggp, q, dinv, b1.reshape(1, D), W2)
    aggp = _agg_call(q, idx).reshape(NC, NPAD, D)
    qp = _mid3(aggp, q, dinv, b2.reshape(1, D), W3)
    ap = [_agg_call(qpj, idx).reshape(NC, NPAD, D) for qpj in qp]

    wc_pad = jnp.pad(Wc, ((0, 0), (0, D - Wc.shape[1])))
    bc_pad = jnp.pad(bc, (0, D - bc.shape[0])).reshape(1, D)
    out = _head(ap[0], ap[1], ap[2], ap[3], qp[0], qp[1], qp[2], qp[3],
                dinv, b3.reshape(1, D_INT),
                Wi, bi.reshape(1, D_INT), wc_pad, bc_pad)
    return out[:, :Wc.shape[1]]


# speedup vs baseline: 16.2528x; 1.0947x over previous
import functools

import jax
import jax.numpy as jnp
from jax import lax
from jax.experimental import pallas as pl
from jax.experimental.pallas import tpu as pltpu
from jax.experimental.pallas import tpu_sc as plsc

N = 10000
E = 320000
D = 128
D_INT = 512

NC = 2
NS = 16
NW = NC * NS
STRIPE = 640
NPAD = NS * STRIPE
EPT = E // NW
C = 100
ITERS = EPT // C
BI = 20
BLOCKS = ITERS // BI
CD = 80
ITERS_D = EPT // CD

_mesh = plsc.VectorSubcoreMesh(
    core_axis_name="c", subcore_axis_name="s", num_cores=NC, num_subcores=NS)

_Z16 = functools.partial(jnp.zeros, (16,), jnp.float32)


def _deg_body(dst_hbm, out_hbm, didx_all, ones_v, zbuf, deg_sh):
    c = lax.axis_index("c")
    s = lax.axis_index("s")

    def fill_ones(k, carry):
        ones_v[pl.ds(k * 16, 16)] = jnp.full((16,), 1.0, jnp.float32)
        return carry
    lax.fori_loop(0, CD // 16, fill_ones, 0)

    def fill_zero(k, carry):
        zbuf[pl.ds(k * 16, 16)] = _Z16()
        return carry
    lax.fori_loop(0, STRIPE // 16, fill_zero, 0)

    pltpu.sync_copy(dst_hbm.at[c * NS + s], didx_all)
    pltpu.sync_copy(zbuf, deg_sh.at[pl.ds(s * STRIPE, STRIPE)])
    plsc.subcore_barrier()

    def step(i, carry):
        pltpu.sync_copy(ones_v, deg_sh.at[didx_all.at[i]], add=True)
        return carry
    lax.fori_loop(0, ITERS_D, step, 0)

    plsc.subcore_barrier()
    pltpu.sync_copy(deg_sh.at[pl.ds(s * STRIPE, STRIPE)],
                    out_hbm.at[pl.ds(c * NPAD + s * STRIPE, STRIPE)])


_deg_call = pl.kernel(
    _deg_body,
    out_type=jax.ShapeDtypeStruct((NC * NPAD,), jnp.float32),
    mesh=_mesh,
    scratch_types=[
        pltpu.VMEM((ITERS_D, CD), jnp.int32),
        pltpu.VMEM((CD,), jnp.float32),
        pltpu.VMEM((STRIPE,), jnp.float32),
        pltpu.VMEM_SHARED((NPAD,), jnp.float32),
    ],
)


def _agg_body(g_hbm, idx_hbm, out_hbm, ix, rows0, rows1, sem_g, sem_i, acc):
    c = lax.axis_index("c")
    s = lax.axis_index("s")
    wid = c * NS + s

    def _ixw(b):
        sl = lax.rem(b, 2)
        pltpu.make_async_copy(idx_hbm.at[wid, b], ix.at[sl], sem_i).wait()

    def zrow(j, carry):
        def zcol(k, carry2):
            rows1[j, pl.ds(k * 16, 16)] = _Z16()
            return carry2
        return lax.fori_loop(0, D // 16, zcol, carry)
    lax.fori_loop(0, C, zrow, 0)

    pltpu.async_copy(idx_hbm.at[wid, 0], ix.at[0], sem_i)
    pltpu.async_copy(idx_hbm.at[wid, 1], ix.at[1], sem_i)

    def zstripe(m, carry):
        pltpu.sync_copy(rows1, acc.at[pl.ds(s * STRIPE + m * C, C)])
        return carry
    lax.fori_loop(0, STRIPE // C, zstripe, 0)
    _REM = STRIPE - (STRIPE // C) * C
    pltpu.sync_copy(rows1.at[pl.ds(0, _REM)],
                    acc.at[pl.ds(s * STRIPE + (STRIPE // C) * C, _REM)])

    _ixw(0)
    pltpu.async_copy(g_hbm.at[ix.at[0, 0, 0]], rows0, sem_g)
    plsc.subcore_barrier()

    def step(i, carry):
        k = 2 * i
        b = lax.div(k, BI)
        j = lax.rem(k, BI)
        sl = lax.rem(b, 2)
        osl = lax.rem(b + 1, 2)

        pltpu.make_async_copy(g_hbm.at[ix.at[sl, j, 0]], rows0, sem_g).wait()
        pltpu.async_copy(g_hbm.at[ix.at[sl, j + 1, 0]], rows1, sem_g)
        pltpu.sync_copy(rows0, acc.at[ix.at[sl, j, 1]], add=True)

        @pl.when(j == BI - 2)
        def _():
            _ixw(b + 1)

        pltpu.make_async_copy(g_hbm.at[ix.at[sl, j + 1, 0]], rows1, sem_g).wait()
        nb = lax.div(k + 2, BI)
        nj = lax.rem(k + 2, BI)
        nsl = lax.rem(nb, 2)
        pltpu.async_copy(g_hbm.at[ix.at[nsl, nj, 0]], rows0, sem_g)
        pltpu.sync_copy(rows1, acc.at[ix.at[sl, j + 1, 1]], add=True)

        @pl.when((j == BI - 2) & (b + 2 <= BLOCKS))
        def _():
            pltpu.async_copy(idx_hbm.at[wid, b + 2], ix.at[sl], sem_i)
        return carry
    lax.fori_loop(0, ITERS // 2, step, 0)

    pltpu.make_async_copy(g_hbm.at[ix.at[lax.rem(BLOCKS, 2), 0, 0]],
                          rows0, sem_g).wait()

    plsc.subcore_barrier()
    pltpu.sync_copy(acc.at[pl.ds(s * STRIPE, STRIPE)],
                    out_hbm.at[pl.ds(c * NPAD + s * STRIPE, STRIPE)])


_agg_call = pl.kernel(
    _agg_body,
    out_type=jax.ShapeDtypeStruct((NC * NPAD, D), jnp.float32),
    mesh=_mesh,
    scratch_types=[
        pltpu.VMEM((2, BI, 2, C), jnp.int32),
        pltpu.VMEM((C, D), jnp.float32),
        pltpu.VMEM((C, D), jnp.float32),
        pltpu.SemaphoreType.DMA,
        pltpu.SemaphoreType.DMA,
        pltpu.VMEM_SHARED((NPAD, D), jnp.float32),
    ],
)


R = 1000
G = N // R


def _pre_body(x_ref, dinv_ref, w_ref, o_ref):
    xw = jnp.dot(x_ref[...], w_ref[...], preferred_element_type=jnp.float32)
    o_ref[...] = xw * dinv_ref[...]


_pre = pl.pallas_call(
    _pre_body,
    grid=(G,),
    in_specs=[
        pl.BlockSpec((R, D), lambda i: (i, 0)),
        pl.BlockSpec((R, 1), lambda i: (i, 0)),
        pl.BlockSpec((D, D), lambda i: (0, 0)),
    ],
    out_specs=pl.BlockSpec((R, D), lambda i: (i, 0)),
    out_shape=jax.ShapeDtypeStruct((N, D), jnp.float32),
)


def _mid_body(aggp_ref, q_ref, dinv_ref, b_ref, w_ref, o_ref):
    dv = dinv_ref[...]
    h = (aggp_ref[0] + aggp_ref[1] + q_ref[...]) * dv + b_ref[...]
    h = jnp.maximum(h, 0.0)
    o_ref[...] = jnp.dot(h, w_ref[...], preferred_element_type=jnp.float32) * dv


_mid = pl.pallas_call(
    _mid_body,
    grid=(G,),
    in_specs=[
        pl.BlockSpec((NC, R, D), lambda i: (0, i, 0)),
        pl.BlockSpec((R, D), lambda i: (i, 0)),
        pl.BlockSpec((R, 1), lambda i: (i, 0)),
        pl.BlockSpec((1, D), lambda i: (0, 0)),
        pl.BlockSpec((D, D), lambda i: (0, 0)),
    ],
    out_specs=pl.BlockSpec((R, D), lambda i: (i, 0)),
    out_shape=jax.ShapeDtypeStruct((N, D), jnp.float32),
)


def _mid3_body(aggp_ref, q_ref, dinv_ref, b_ref, w_ref,
               o0_ref, o1_ref, o2_ref, o3_ref):
    dv = dinv_ref[...]
    h = (aggp_ref[0] + aggp_ref[1] + q_ref[...]) * dv + b_ref[...]
    h = jnp.maximum(h, 0.0)
    q3 = jnp.dot(h, w_ref[...], preferred_element_type=jnp.float32) * dv
    o0_ref[...] = q3[:, :D]
    o1_ref[...] = q3[:, D:2 * D]
    o2_ref[...] = q3[:, 2 * D:3 * D]
    o3_ref[...] = q3[:, 3 * D:]


_mid3 = pl.pallas_call(
    _mid3_body,
    grid=(G,),
    in_specs=[
        pl.BlockSpec((NC, R, D), lambda i: (0, i, 0)),
        pl.BlockSpec((R, D), lambda i: (i, 0)),
        pl.BlockSpec((R, 1), lambda i: (i, 0)),
        pl.BlockSpec((1, D), lambda i: (0, 0)),
        pl.BlockSpec((D, D_INT), lambda i: (0, 0)),
    ],
    out_specs=tuple(pl.BlockSpec((R, D), lambda i: (i, 0)) for _ in range(4)),
    out_shape=tuple(jax.ShapeDtypeStruct((N, D), jnp.float32) for _ in range(4)),
)


def _head_body(a0_ref, a1_ref, a2_ref, a3_ref, q0_ref, q1_ref, q2_ref, q3_ref,
               dinv_ref, b3_ref, wi_ref, bi_ref, wc_ref, bc_ref, o_ref):
    dv = dinv_ref[...]
    t = jnp.concatenate(
        [(a0_ref[0] + a0_ref[1] + q0_ref[...]),
         (a1_ref[0] + a1_ref[1] + q1_ref[...]),
         (a2_ref[0] + a2_ref[1] + q2_ref[...]),
         (a3_ref[0] + a3_ref[1] + q3_ref[...])], axis=1) * dv
    h3 = jnp.maximum(t + b3_ref[...], 0.0)
    h4 = jnp.dot(h3, wi_ref[...], preferred_element_type=jnp.float32) + bi_ref[...]
    h4 = jnp.maximum(h4, 0.0)
    o_ref[...] = jnp.dot(h4, wc_ref[...], preferred_element_type=jnp.float32) + bc_ref[...]


_head = pl.pallas_call(
    _head_body,
    grid=(G,),
    in_specs=(
        [pl.BlockSpec((NC, R, D), lambda i: (0, i, 0)) for _ in range(4)]
        + [pl.BlockSpec((R, D), lambda i: (i, 0)) for _ in range(4)]
        + [
            pl.BlockSpec((R, 1), lambda i: (i, 0)),
            pl.BlockSpec((1, D_INT), lambda i: (0, 0)),
            pl.BlockSpec((D_INT, D_INT), lambda i: (0, 0)),
            pl.BlockSpec((1, D_INT), lambda i: (0, 0)),
            pl.BlockSpec((D_INT, D), lambda i: (0, 0)),
            pl.BlockSpec((1, D), lambda i: (0, 0)),
        ]
    ),
    out_specs=pl.BlockSpec((R, D), lambda i: (i, 0)),
    out_shape=jax.ShapeDtypeStruct((N, D), jnp.float32),
)


def kernel(x, edge_index, W1, b1, W2, b2, W3, b3, Wi, bi, Wc, bc):
    dst = edge_index[1].reshape(NW, ITERS_D, CD)
    idx = edge_index.reshape(2, NW, ITERS, C).transpose(1, 2, 0, 3)
    idx = idx.reshape(NW, BLOCKS, BI, 2, C)
    idx = jnp.concatenate([idx, idx[:, :1]], axis=1)

    degp = _deg_call(dst).reshape(NC, NPAD)
    deg = degp[0, :N] + degp[1, :N] + 1.0
    dinv = lax.rsqrt(deg).reshape(N, 1)

    q = _pre(x, dinv, W1)
    aggp = _agg_call(q, idx).reshape(NC, NPAD, D)
    q = _mid(aggp, q, dinv, b1.reshape(1, D), W2)
    aggp = _agg_call(q, idx).reshape(NC, NPAD, D)
    qp = _mid3(aggp, q, dinv, b2.reshape(1, D), W3)
    ap = [_agg_call(qpj, idx).reshape(NC, NPAD, D) for qpj in qp]

    wc_pad = jnp.pad(Wc, ((0, 0), (0, D - Wc.shape[1])))
    bc_pad = jnp.pad(bc, (0, D - bc.shape[0])).reshape(1, D)
    out = _head(ap[0], ap[1], ap[2], ap[3], qp[0], qp[1], qp[2], qp[3],
                dinv, b3.reshape(1, D_INT),
                Wi, bi.reshape(1, D_INT), wc_pad, bc_pad)
    return out[:, :Wc.shape[1]]


# speedup vs baseline: 17.5403x; 1.0792x over previous
import functools

import jax
import jax.numpy as jnp
from jax import lax
from jax.experimental import pallas as pl
from jax.experimental.pallas import tpu as pltpu
from jax.experimental.pallas import tpu_sc as plsc

N = 10000
E = 320000
D = 128
D_INT = 512

NC = 2
NS = 16
NW = NC * NS
STRIPE = 640
NPAD = NS * STRIPE
EPT = E // NW
C = 125
ITERS = EPT // C
BI = 8
BLOCKS = ITERS // BI
CD = 80
ITERS_D = EPT // CD

_mesh = plsc.VectorSubcoreMesh(
    core_axis_name="c", subcore_axis_name="s", num_cores=NC, num_subcores=NS)

_Z16 = functools.partial(jnp.zeros, (16,), jnp.float32)


def _deg_body(dst_hbm, out_hbm, didx_all, ones_v, zbuf, deg_sh):
    c = lax.axis_index("c")
    s = lax.axis_index("s")

    def fill_ones(k, carry):
        ones_v[pl.ds(k * 16, 16)] = jnp.full((16,), 1.0, jnp.float32)
        return carry
    lax.fori_loop(0, CD // 16, fill_ones, 0)

    def fill_zero(k, carry):
        zbuf[pl.ds(k * 16, 16)] = _Z16()
        return carry
    lax.fori_loop(0, STRIPE // 16, fill_zero, 0)

    pltpu.sync_copy(dst_hbm.at[c * NS + s], didx_all)
    pltpu.sync_copy(zbuf, deg_sh.at[pl.ds(s * STRIPE, STRIPE)])
    plsc.subcore_barrier()

    def step(i, carry):
        pltpu.sync_copy(ones_v, deg_sh.at[didx_all.at[i]], add=True)
        return carry
    lax.fori_loop(0, ITERS_D, step, 0)

    plsc.subcore_barrier()
    pltpu.sync_copy(deg_sh.at[pl.ds(s * STRIPE, STRIPE)],
                    out_hbm.at[pl.ds(c * NPAD + s * STRIPE, STRIPE)])


_deg_call = pl.kernel(
    _deg_body,
    out_type=jax.ShapeDtypeStruct((NC * NPAD,), jnp.float32),
    mesh=_mesh,
    scratch_types=[
        pltpu.VMEM((ITERS_D, CD), jnp.int32),
        pltpu.VMEM((CD,), jnp.float32),
        pltpu.VMEM((STRIPE,), jnp.float32),
        pltpu.VMEM_SHARED((NPAD,), jnp.float32),
    ],
)


def _agg_body(g_hbm, idx_hbm, out_hbm, ix, rows0, rows1, sem_g, sem_i, acc):
    c = lax.axis_index("c")
    s = lax.axis_index("s")
    wid = c * NS + s

    def _ixw(b):
        sl = lax.rem(b, 2)
        pltpu.make_async_copy(idx_hbm.at[wid, b], ix.at[sl], sem_i).wait()

    def zrow(j, carry):
        def zcol(k, carry2):
            rows1[j, pl.ds(k * 16, 16)] = _Z16()
            return carry2
        return lax.fori_loop(0, D // 16, zcol, carry)
    lax.fori_loop(0, C, zrow, 0)

    pltpu.async_copy(idx_hbm.at[wid, 0], ix.at[0], sem_i)
    pltpu.async_copy(idx_hbm.at[wid, 1], ix.at[1], sem_i)

    def zstripe(m, carry):
        pltpu.sync_copy(rows1, acc.at[pl.ds(s * STRIPE + m * C, C)])
        return carry
    lax.fori_loop(0, STRIPE // C, zstripe, 0)
    _REM = STRIPE - (STRIPE // C) * C
    pltpu.sync_copy(rows1.at[pl.ds(0, _REM)],
                    acc.at[pl.ds(s * STRIPE + (STRIPE // C) * C, _REM)])

    _ixw(0)
    pltpu.async_copy(g_hbm.at[ix.at[0, 0, 0]], rows0, sem_g)
    plsc.subcore_barrier()

    def step(i, carry):
        k = 2 * i
        b = lax.div(k, BI)
        j = lax.rem(k, BI)
        sl = lax.rem(b, 2)
        osl = lax.rem(b + 1, 2)

        pltpu.make_async_copy(g_hbm.at[ix.at[sl, j, 0]], rows0, sem_g).wait()
        pltpu.async_copy(g_hbm.at[ix.at[sl, j + 1, 0]], rows1, sem_g)
        pltpu.sync_copy(rows0, acc.at[ix.at[sl, j, 1]], add=True)

        @pl.when(j == BI - 2)
        def _():
            _ixw(b + 1)

        pltpu.make_async_copy(g_hbm.at[ix.at[sl, j + 1, 0]], rows1, sem_g).wait()
        nb = lax.div(k + 2, BI)
        nj = lax.rem(k + 2, BI)
        nsl = lax.rem(nb, 2)
        pltpu.async_copy(g_hbm.at[ix.at[nsl, nj, 0]], rows0, sem_g)
        pltpu.sync_copy(rows1, acc.at[ix.at[sl, j + 1, 1]], add=True)

        @pl.when((j == BI - 2) & (b + 2 <= BLOCKS))
        def _():
            pltpu.async_copy(idx_hbm.at[wid, b + 2], ix.at[sl], sem_i)
        return carry
    lax.fori_loop(0, ITERS // 2, step, 0)

    pltpu.make_async_copy(g_hbm.at[ix.at[lax.rem(BLOCKS, 2), 0, 0]],
                          rows0, sem_g).wait()

    plsc.subcore_barrier()
    pltpu.sync_copy(acc.at[pl.ds(s * STRIPE, STRIPE)],
                    out_hbm.at[pl.ds(c * NPAD + s * STRIPE, STRIPE)])


_agg_call = pl.kernel(
    _agg_body,
    out_type=jax.ShapeDtypeStruct((NC * NPAD, D), jnp.float32),
    mesh=_mesh,
    scratch_types=[
        pltpu.VMEM((2, BI, 2, C), jnp.int32),
        pltpu.VMEM((C, D), jnp.float32),
        pltpu.VMEM((C, D), jnp.float32),
        pltpu.SemaphoreType.DMA,
        pltpu.SemaphoreType.DMA,
        pltpu.VMEM_SHARED((NPAD, D), jnp.float32),
    ],
)


R = 1000
G = N // R


def _pre_body(x_ref, dinv_ref, w_ref, o_ref):
    xw = jnp.dot(x_ref[...], w_ref[...], preferred_element_type=jnp.float32)
    o_ref[...] = xw * dinv_ref[...]


_pre = pl.pallas_call(
    _pre_body,
    grid=(G,),
    in_specs=[
        pl.BlockSpec((R, D), lambda i: (i, 0)),
        pl.BlockSpec((R, 1), lambda i: (i, 0)),
        pl.BlockSpec((D, D), lambda i: (0, 0)),
    ],
    out_specs=pl.BlockSpec((R, D), lambda i: (i, 0)),
    out_shape=jax.ShapeDtypeStruct((N, D), jnp.float32),
)


def _mid_body(aggp_ref, q_ref, dinv_ref, b_ref, w_ref, o_ref):
    dv = dinv_ref[...]
    h = (aggp_ref[0] + aggp_ref[1] + q_ref[...]) * dv + b_ref[...]
    h = jnp.maximum(h, 0.0)
    o_ref[...] = jnp.dot(h, w_ref[...], preferred_element_type=jnp.float32) * dv


_mid = pl.pallas_call(
    _mid_body,
    grid=(G,),
    in_specs=[
        pl.BlockSpec((NC, R, D), lambda i: (0, i, 0)),
        pl.BlockSpec((R, D), lambda i: (i, 0)),
        pl.BlockSpec((R, 1), lambda i: (i, 0)),
        pl.BlockSpec((1, D), lambda i: (0, 0)),
        pl.BlockSpec((D, D), lambda i: (0, 0)),
    ],
    out_specs=pl.BlockSpec((R, D), lambda i: (i, 0)),
    out_shape=jax.ShapeDtypeStruct((N, D), jnp.float32),
)


def _mid3_body(aggp_ref, q_ref, dinv_ref, b_ref, w_ref,
               o0_ref, o1_ref, o2_ref, o3_ref):
    dv = dinv_ref[...]
    h = (aggp_ref[0] + aggp_ref[1] + q_ref[...]) * dv + b_ref[...]
    h = jnp.maximum(h, 0.0)
    q3 = jnp.dot(h, w_ref[...], preferred_element_type=jnp.float32) * dv
    o0_ref[...] = q3[:, :D]
    o1_ref[...] = q3[:, D:2 * D]
    o2_ref[...] = q3[:, 2 * D:3 * D]
    o3_ref[...] = q3[:, 3 * D:]


_mid3 = pl.pallas_call(
    _mid3_body,
    grid=(G,),
    in_specs=[
        pl.BlockSpec((NC, R, D), lambda i: (0, i, 0)),
        pl.BlockSpec((R, D), lambda i: (i, 0)),
        pl.BlockSpec((R, 1), lambda i: (i, 0)),
        pl.BlockSpec((1, D), lambda i: (0, 0)),
        pl.BlockSpec((D, D_INT), lambda i: (0, 0)),
    ],
    out_specs=tuple(pl.BlockSpec((R, D), lambda i: (i, 0)) for _ in range(4)),
    out_shape=tuple(jax.ShapeDtypeStruct((N, D), jnp.float32) for _ in range(4)),
)


def _head_body(a0_ref, a1_ref, a2_ref, a3_ref, q0_ref, q1_ref, q2_ref, q3_ref,
               dinv_ref, b3_ref, wi_ref, bi_ref, wc_ref, bc_ref, o_ref):
    dv = dinv_ref[...]
    t = jnp.concatenate(
        [(a0_ref[0] + a0_ref[1] + q0_ref[...]),
         (a1_ref[0] + a1_ref[1] + q1_ref[...]),
         (a2_ref[0] + a2_ref[1] + q2_ref[...]),
         (a3_ref[0] + a3_ref[1] + q3_ref[...])], axis=1) * dv
    h3 = jnp.maximum(t + b3_ref[...], 0.0)
    h4 = jnp.dot(h3, wi_ref[...], preferred_element_type=jnp.float32) + bi_ref[...]
    h4 = jnp.maximum(h4, 0.0)
    o_ref[...] = jnp.dot(h4, wc_ref[...], preferred_element_type=jnp.float32) + bc_ref[...]


_head = pl.pallas_call(
    _head_body,
    grid=(G,),
    in_specs=(
        [pl.BlockSpec((NC, R, D), lambda i: (0, i, 0)) for _ in range(4)]
        + [pl.BlockSpec((R, D), lambda i: (i, 0)) for _ in range(4)]
        + [
            pl.BlockSpec((R, 1), lambda i: (i, 0)),
            pl.BlockSpec((1, D_INT), lambda i: (0, 0)),
            pl.BlockSpec((D_INT, D_INT), lambda i: (0, 0)),
            pl.BlockSpec((1, D_INT), lambda i: (0, 0)),
            pl.BlockSpec((D_INT, D), lambda i: (0, 0)),
            pl.BlockSpec((1, D), lambda i: (0, 0)),
        ]
    ),
    out_specs=pl.BlockSpec((R, D), lambda i: (i, 0)),
    out_shape=jax.ShapeDtypeStruct((N, D), jnp.float32),
)


def kernel(x, edge_index, W1, b1, W2, b2, W3, b3, Wi, bi, Wc, bc):
    dst = edge_index[1].reshape(NW, ITERS_D, CD)
    idx = edge_index.reshape(2, NW, ITERS, C).transpose(1, 2, 0, 3)
    idx = idx.reshape(NW, BLOCKS, BI, 2, C)
    idx = jnp.concatenate([idx, idx[:, :1]], axis=1)

    degp = _deg_call(dst).reshape(NC, NPAD)
    deg = degp[0, :N] + degp[1, :N] + 1.0
    dinv = lax.rsqrt(deg).reshape(N, 1)

    q = _pre(x, dinv, W1)
    aggp = _agg_call(q, idx).reshape(NC, NPAD, D)
    q = _mid(aggp, q, dinv, b1.reshape(1, D), W2)
    aggp = _agg_call(q, idx).reshape(NC, NPAD, D)
    qp = _mid3(aggp, q, dinv, b2.reshape(1, D), W3)
    ap = [_agg_call(qpj, idx).reshape(NC, NPAD, D) for qpj in qp]

    wc_pad = jnp.pad(Wc, ((0, 0), (0, D - Wc.shape[1])))
    bc_pad = jnp.pad(bc, (0, D - bc.shape[0])).reshape(1, D)
    out = _head(ap[0], ap[1], ap[2], ap[3], qp[0], qp[1], qp[2], qp[3],
                dinv, b3.reshape(1, D_INT),
                Wi, bi.reshape(1, D_INT), wc_pad, bc_pad)
    return out[:, :Wc.shape[1]]


# speedup vs baseline: 19.1565x; 1.0921x over previous
import functools

import jax
import jax.numpy as jnp
from jax import lax
from jax.experimental import pallas as pl
from jax.experimental.pallas import tpu as pltpu
from jax.experimental.pallas import tpu_sc as plsc

N = 10000
E = 320000
D = 128
D_INT = 512

NC = 2
NS = 16
NW = NC * NS
STRIPE = 640
NPAD = NS * STRIPE
EPT = E // NW
C = 125
ITERS = EPT // C
BI = 8
BLOCKS = ITERS // BI
CD = 80
ITERS_D = EPT // CD

_mesh = plsc.VectorSubcoreMesh(
    core_axis_name="c", subcore_axis_name="s", num_cores=NC, num_subcores=NS)

_Z16 = functools.partial(jnp.zeros, (16,), jnp.float32)


def _deg_body(dst_hbm, out_hbm, didx_all, ones_v, zbuf, deg_sh):
    c = lax.axis_index("c")
    s = lax.axis_index("s")

    def fill_ones(k, carry):
        ones_v[pl.ds(k * 16, 16)] = jnp.full((16,), 1.0, jnp.float32)
        return carry
    lax.fori_loop(0, CD // 16, fill_ones, 0)

    def fill_zero(k, carry):
        zbuf[pl.ds(k * 16, 16)] = _Z16()
        return carry
    lax.fori_loop(0, STRIPE // 16, fill_zero, 0)

    pltpu.sync_copy(dst_hbm.at[c * NS + s], didx_all)
    pltpu.sync_copy(zbuf, deg_sh.at[pl.ds(s * STRIPE, STRIPE)])
    plsc.subcore_barrier()

    def step(i, carry):
        pltpu.sync_copy(ones_v, deg_sh.at[didx_all.at[i]], add=True)
        return carry
    lax.fori_loop(0, ITERS_D, step, 0)

    plsc.subcore_barrier()
    pltpu.sync_copy(deg_sh.at[pl.ds(s * STRIPE, STRIPE)],
                    out_hbm.at[pl.ds(c * NPAD + s * STRIPE, STRIPE)])


_deg_call = pl.kernel(
    _deg_body,
    out_type=jax.ShapeDtypeStruct((NC * NPAD,), jnp.float32),
    mesh=_mesh,
    scratch_types=[
        pltpu.VMEM((ITERS_D, CD), jnp.int32),
        pltpu.VMEM((CD,), jnp.float32),
        pltpu.VMEM((STRIPE,), jnp.float32),
        pltpu.VMEM_SHARED((NPAD,), jnp.float32),
    ],
)


def _make_agg(dtype):
  def _agg_body(g_hbm, idx_hbm, out_hbm, ix, rows0, rows1, sem_g, sem_i, acc):
        c = lax.axis_index("c")
        s = lax.axis_index("s")
        wid = c * NS + s

        def _ixw(b):
            sl = lax.rem(b, 2)
            pltpu.make_async_copy(idx_hbm.at[wid, b], ix.at[sl], sem_i).wait()

        def zrow(j, carry):
            def zcol(k, carry2):
                rows1[j, pl.ds(k * 16, 16)] = jnp.zeros((16,), dtype)
                return carry2
            return lax.fori_loop(0, D // 16, zcol, carry)
        lax.fori_loop(0, C, zrow, 0)

        pltpu.async_copy(idx_hbm.at[wid, 0], ix.at[0], sem_i)
        pltpu.async_copy(idx_hbm.at[wid, 1], ix.at[1], sem_i)

        def zstripe(m, carry):
            pltpu.sync_copy(rows1, acc.at[pl.ds(s * STRIPE + m * C, C)])
            return carry
        lax.fori_loop(0, STRIPE // C, zstripe, 0)
        _REM = STRIPE - (STRIPE // C) * C
        pltpu.sync_copy(rows1.at[pl.ds(0, _REM)],
                        acc.at[pl.ds(s * STRIPE + (STRIPE // C) * C, _REM)])

        _ixw(0)
        pltpu.async_copy(g_hbm.at[ix.at[0, 0, 0]], rows0, sem_g)
        plsc.subcore_barrier()

        def step(i, carry):
            k = 2 * i
            b = lax.div(k, BI)
            j = lax.rem(k, BI)
            sl = lax.rem(b, 2)
            osl = lax.rem(b + 1, 2)

            pltpu.make_async_copy(g_hbm.at[ix.at[sl, j, 0]], rows0, sem_g).wait()
            pltpu.async_copy(g_hbm.at[ix.at[sl, j + 1, 0]], rows1, sem_g)
            pltpu.sync_copy(rows0, acc.at[ix.at[sl, j, 1]], add=True)

            @pl.when(j == BI - 2)
            def _():
                _ixw(b + 1)

            pltpu.make_async_copy(g_hbm.at[ix.at[sl, j + 1, 0]], rows1, sem_g).wait()
            nb = lax.div(k + 2, BI)
            nj = lax.rem(k + 2, BI)
            nsl = lax.rem(nb, 2)
            pltpu.async_copy(g_hbm.at[ix.at[nsl, nj, 0]], rows0, sem_g)
            pltpu.sync_copy(rows1, acc.at[ix.at[sl, j + 1, 1]], add=True)

            @pl.when((j == BI - 2) & (b + 2 <= BLOCKS))
            def _():
                pltpu.async_copy(idx_hbm.at[wid, b + 2], ix.at[sl], sem_i)
            return carry
        lax.fori_loop(0, ITERS // 2, step, 0)

        pltpu.make_async_copy(g_hbm.at[ix.at[lax.rem(BLOCKS, 2), 0, 0]],
                              rows0, sem_g).wait()

        plsc.subcore_barrier()
        pltpu.sync_copy(acc.at[pl.ds(s * STRIPE, STRIPE)],
                        out_hbm.at[pl.ds(c * NPAD + s * STRIPE, STRIPE)])


  return pl.kernel(
        _agg_body,
      out_type=jax.ShapeDtypeStruct((NC * NPAD, D), dtype),
      mesh=_mesh,
      scratch_types=[
          pltpu.VMEM((2, BI, 2, C), jnp.int32),
          pltpu.VMEM((C, D), dtype),
          pltpu.VMEM((C, D), dtype),
          pltpu.SemaphoreType.DMA,
          pltpu.SemaphoreType.DMA,
          pltpu.VMEM_SHARED((NPAD, D), dtype),
      ],
  )


_agg_call = _make_agg(jnp.float32)
_agg_i32 = _make_agg(jnp.int32)


R = 1000
G = N // R


def _pre_body(x_ref, dinv_ref, w_ref, o_ref):
    xw = jnp.dot(x_ref[...], w_ref[...], preferred_element_type=jnp.float32)
    o_ref[...] = xw * dinv_ref[...]


_pre = pl.pallas_call(
    _pre_body,
    grid=(G,),
    in_specs=[
        pl.BlockSpec((R, D), lambda i: (i, 0)),
        pl.BlockSpec((R, 1), lambda i: (i, 0)),
        pl.BlockSpec((D, D), lambda i: (0, 0)),
    ],
    out_specs=pl.BlockSpec((R, D), lambda i: (i, 0)),
    out_shape=jax.ShapeDtypeStruct((N, D), jnp.float32),
)


def _mid_body(aggp_ref, q_ref, dinv_ref, b_ref, w_ref, o_ref):
    dv = dinv_ref[...]
    h = (aggp_ref[0] + aggp_ref[1] + q_ref[...]) * dv + b_ref[...]
    h = jnp.maximum(h, 0.0)
    o_ref[...] = jnp.dot(h, w_ref[...], preferred_element_type=jnp.float32) * dv


_mid = pl.pallas_call(
    _mid_body,
    grid=(G,),
    in_specs=[
        pl.BlockSpec((NC, R, D), lambda i: (0, i, 0)),
        pl.BlockSpec((R, D), lambda i: (i, 0)),
        pl.BlockSpec((R, 1), lambda i: (i, 0)),
        pl.BlockSpec((1, D), lambda i: (0, 0)),
        pl.BlockSpec((D, D), lambda i: (0, 0)),
    ],
    out_specs=pl.BlockSpec((R, D), lambda i: (i, 0)),
    out_shape=jax.ShapeDtypeStruct((N, D), jnp.float32),
)


QMAX = 127
SHIFT = 65536


def _l3a_body(aggp_ref, q_ref, dinv_ref, b_ref, h_ref, o_ref):
    dv = dinv_ref[...]
    h = (aggp_ref[0] + aggp_ref[1] + q_ref[...]) * dv + b_ref[...]
    h = jnp.maximum(h, 0.0)
    h_ref[...] = h
    o_ref[...] = h * dv


_l3a = pl.pallas_call(
    _l3a_body,
    grid=(G,),
    in_specs=[
        pl.BlockSpec((NC, R, D), lambda i: (0, i, 0)),
        pl.BlockSpec((R, D), lambda i: (i, 0)),
        pl.BlockSpec((R, 1), lambda i: (i, 0)),
        pl.BlockSpec((1, D), lambda i: (0, 0)),
    ],
    out_specs=(pl.BlockSpec((R, D), lambda i: (i, 0)),
               pl.BlockSpec((R, D), lambda i: (i, 0))),
    out_shape=(jax.ShapeDtypeStruct((N, D), jnp.float32),
               jax.ShapeDtypeStruct((N, D), jnp.float32)),
)


def _l3b_body(aggp_ref, q_ref, h2_ref, dinv_ref, w3_ref, u_ref, e_ref, bm_ref):
    dv = dinv_ref[...]
    ah2 = (aggp_ref[0] + aggp_ref[1] + q_ref[...]) * dv
    u_ref[...] = jnp.dot(ah2, w3_ref[...], preferred_element_type=jnp.float32,
                         precision=lax.Precision.HIGHEST)
    h2 = h2_ref[...]
    xw3 = jnp.dot(h2, w3_ref[...], preferred_element_type=jnp.float32)
    v = jnp.dot(h2, w3_ref[...], preferred_element_type=jnp.float32,
                precision=lax.Precision.HIGHEST)
    e = (xw3 - v) * dv
    e_ref[...] = e
    bm_ref[...] = jnp.broadcast_to(jnp.max(jnp.abs(e)), (8, D))


_l3b = pl.pallas_call(
    _l3b_body,
    grid=(G,),
    in_specs=[
        pl.BlockSpec((NC, R, D), lambda i: (0, i, 0)),
        pl.BlockSpec((R, D), lambda i: (i, 0)),
        pl.BlockSpec((R, D), lambda i: (i, 0)),
        pl.BlockSpec((R, 1), lambda i: (i, 0)),
        pl.BlockSpec((D, D_INT), lambda i: (0, 0)),
    ],
    out_specs=(pl.BlockSpec((R, D_INT), lambda i: (i, 0)),
               pl.BlockSpec((R, D_INT), lambda i: (i, 0)),
               pl.BlockSpec((8, D), lambda i: (i, 0))),
    out_shape=(jax.ShapeDtypeStruct((N, D_INT), jnp.float32),
               jax.ShapeDtypeStruct((N, D_INT), jnp.float32),
               jax.ShapeDtypeStruct((G * 8, D), jnp.float32)),
)


def _pack_body(e_ref, inv_ref, p0_ref, p1_ref):
    r = e_ref[...] * inv_ref[...]
    r = r + jnp.where(r >= 0, 0.5, -0.5)
    q = jnp.clip(r.astype(jnp.int32), -QMAX, QMAX)
    p = q[:, :2 * D] * SHIFT + q[:, 2 * D:]
    p0_ref[...] = p[:, :D]
    p1_ref[...] = p[:, D:]


_pack = pl.pallas_call(
    _pack_body,
    grid=(G,),
    in_specs=[
        pl.BlockSpec((R, D_INT), lambda i: (i, 0)),
        pl.BlockSpec((1, 1), lambda i: (0, 0)),
    ],
    out_specs=(pl.BlockSpec((R, D), lambda i: (i, 0)),
               pl.BlockSpec((R, D), lambda i: (i, 0))),
    out_shape=(jax.ShapeDtypeStruct((N, D), jnp.int32),
               jax.ShapeDtypeStruct((N, D), jnp.int32)),
)


def _head_body(pa0_ref, pa1_ref, p0_ref, p1_ref, u_ref, dinv_ref, sa_ref,
               b3_ref, wi_ref, bi_ref, wc_ref, bc_ref, o_ref):
    dv = dinv_ref[...]
    s0 = pa0_ref[0] + pa0_ref[1] + p0_ref[...]
    s1 = pa1_ref[0] + pa1_ref[1] + p1_ref[...]
    hi0 = lax.shift_right_arithmetic(s0 + SHIFT // 2, 16)
    hi1 = lax.shift_right_arithmetic(s1 + SHIFT // 2, 16)
    lo0 = s0 - hi0 * SHIFT
    lo1 = s1 - hi1 * SHIFT
    eagg = jnp.concatenate([hi0, hi1, lo0, lo1], axis=1).astype(jnp.float32)
    aerr = eagg * sa_ref[...] * dv
    h3 = jnp.maximum(u_ref[...] + aerr + b3_ref[...], 0.0)
    h4 = jnp.dot(h3, wi_ref[...], preferred_element_type=jnp.float32) + bi_ref[...]
    h4 = jnp.maximum(h4, 0.0)
    o_ref[...] = jnp.dot(h4, wc_ref[...], preferred_element_type=jnp.float32) + bc_ref[...]


_head = pl.pallas_call(
    _head_body,
    grid=(G,),
    in_specs=[
        pl.BlockSpec((NC, R, D), lambda i: (0, i, 0)),
        pl.BlockSpec((NC, R, D), lambda i: (0, i, 0)),
        pl.BlockSpec((R, D), lambda i: (i, 0)),
        pl.BlockSpec((R, D), lambda i: (i, 0)),
        pl.BlockSpec((R, D_INT), lambda i: (i, 0)),
        pl.BlockSpec((R, 1), lambda i: (i, 0)),
        pl.BlockSpec((1, 1), lambda i: (0, 0)),
        pl.BlockSpec((1, D_INT), lambda i: (0, 0)),
        pl.BlockSpec((D_INT, D_INT), lambda i: (0, 0)),
        pl.BlockSpec((1, D_INT), lambda i: (0, 0)),
        pl.BlockSpec((D_INT, D), lambda i: (0, 0)),
        pl.BlockSpec((1, D), lambda i: (0, 0)),
    ],
    out_specs=pl.BlockSpec((R, D), lambda i: (i, 0)),
    out_shape=jax.ShapeDtypeStruct((N, D), jnp.float32),
)


def kernel(x, edge_index, W1, b1, W2, b2, W3, b3, Wi, bi, Wc, bc):
    dst = edge_index[1].reshape(NW, ITERS_D, CD)
    idx = edge_index.reshape(2, NW, ITERS, C).transpose(1, 2, 0, 3)
    idx = idx.reshape(NW, BLOCKS, BI, 2, C)
    idx = jnp.concatenate([idx, idx[:, :1]], axis=1)

    degp = _deg_call(dst).reshape(NC, NPAD)
    deg = degp[0, :N] + degp[1, :N] + 1.0
    dinv = lax.rsqrt(deg).reshape(N, 1)

    q = _pre(x, dinv, W1)
    aggp = _agg_call(q, idx).reshape(NC, NPAD, D)
    q = _mid(aggp, q, dinv, b1.reshape(1, D), W2)
    aggp = _agg_call(q, idx).reshape(NC, NPAD, D)
    h2, qh2 = _l3a(aggp, q, dinv, b2.reshape(1, D))
    aggp = _agg_call(qh2, idx).reshape(NC, NPAD, D)
    u, e, bm = _l3b(aggp, qh2, h2, dinv, W3)

    amax = jnp.maximum(jnp.max(bm), 1e-30)
    sa = (amax / QMAX).reshape(1, 1)
    inv_sa = (QMAX / amax).reshape(1, 1)
    p0, p1 = _pack(e, inv_sa)
    pa0 = _agg_i32(p0, idx).reshape(NC, NPAD, D)
    pa1 = _agg_i32(p1, idx).reshape(NC, NPAD, D)

    wc_pad = jnp.pad(Wc, ((0, 0), (0, D - Wc.shape[1])))
    bc_pad = jnp.pad(bc, (0, D - bc.shape[0])).reshape(1, D)
    out = _head(pa0, pa1, p0, p1, u, dinv, sa, b3.reshape(1, D_INT),
                Wi, bi.reshape(1, D_INT), wc_pad, bc_pad)
    return out[:, :Wc.shape[1]]
